# baseline (speedup 1.0000x reference)
import functools

import numpy as np
import jax
import jax.numpy as jnp
from jax import lax
from jax.experimental import pallas as pl
from jax.experimental.pallas import tpu as pltpu
from jax.experimental.pallas import tpu_sc as plsc

N = 10000
E = 320000
D = 128
H = 64
G = 64
C = 16

NC = 2
NS = 16
NW = NC * NS
CHUNK = 128
CPW = 80
NCH = NW * CPW
EP = NCH * CHUNK
NP = 10240
ZR = NP // NS

_IT = np.arange(EP - E, dtype=np.int32)
_PAD_S = _IT % N
_PAD_D = N + _IT % (NP - N)


NB = 8
NOUT = CPW // NB


def _sc_agg_body(y_hbm, src_hbm, dst_hbm, z_hbm, out_hbm, sidx, didx, rows,
                 gsem, ssem, agg):
    cid = lax.axis_index("c")
    sid = lax.axis_index("s")
    pltpu.sync_copy(z_hbm, agg.at[pl.ds(sid * ZR, ZR)])
    base = (cid * NS + sid) * CPW
    pltpu.sync_copy(src_hbm.at[pl.ds(base, CPW)], sidx)
    pltpu.sync_copy(dst_hbm.at[pl.ds(base, CPW)], didx)
    plsc.subcore_barrier()

    @pl.loop(0, NOUT)
    def _outer(t):
        descs = []
        for k in range(NB):
            @pl.when(t > 0)
            def _drain():
                pltpu.make_async_copy(z_hbm.at[pl.ds(0, CHUNK)],
                                      rows.at[k], ssem.at[k]).wait()
            descs.append(pltpu.async_copy(y_hbm.at[sidx.at[t * NB + k]],
                                          rows.at[k], gsem.at[k]))
        for k in range(NB):
            descs[k].wait()
            pltpu.async_copy(rows.at[k], agg.at[didx.at[t * NB + k]],
                             ssem.at[k], add=True)

    for k in range(NB):
        pltpu.make_async_copy(z_hbm.at[pl.ds(0, CHUNK)],
                              rows.at[k], ssem.at[k]).wait()
    plsc.subcore_barrier()
    pltpu.sync_copy(agg.at[pl.ds(sid * ZR, ZR)],
                    out_hbm.at[cid, pl.ds(sid * ZR, ZR), pl.ds(0, H)])


@functools.cache
def _sc_agg():
    mesh = plsc.VectorSubcoreMesh(core_axis_name="c", subcore_axis_name="s",
                                  num_cores=NC, num_subcores=NS)
    return pl.kernel(
        _sc_agg_body,
        out_type=jax.ShapeDtypeStruct((NC, NP, 2 * H), jnp.float32),
        mesh=mesh,
        compiler_params=pltpu.CompilerParams(use_tc_tiling_on_sc=False),
        scratch_types=[
            pltpu.VMEM((CPW, CHUNK), jnp.int32),
            pltpu.VMEM((CPW, CHUNK), jnp.int32),
            pltpu.VMEM((NB, CHUNK, H), jnp.float32),
            pltpu.SemaphoreType.DMA((NB,)),
            pltpu.SemaphoreType.DMA((NB,)),
            pltpu.VMEM_SHARED((NP, H), jnp.float32),
        ],
    )


def _agg_partials(y, src2, dst2, zblk):
    return _sc_agg()(y, src2, dst2, zblk)


def _proj_body(x_ref, w_ref, o_ref):
    o_ref[...] = jnp.dot(x_ref[...], w_ref[...],
                         preferred_element_type=jnp.float32)


def _proj(x, w):
    return pl.pallas_call(
        _proj_body,
        out_shape=jax.ShapeDtypeStruct((N, H), jnp.float32),
    )(x, w)


def _layer_math(y_ref, parts_ref, eps_ref, b1_ref, w2_ref, b2_ref, g_ref,
                be_ref):
    agg = parts_ref[0, :N, 0:H] + parts_ref[1, :N, 0:H]
    t = jnp.maximum(
        (1.0 + eps_ref[0, 0]) * y_ref[...] + agg + b1_ref[0, :], 0.0)
    hraw = jnp.maximum(
        jnp.dot(t, w2_ref[...], preferred_element_type=jnp.float32)
        + b2_ref[0, :], 0.0)
    mu = jnp.mean(hraw, axis=0, keepdims=True)
    hc = hraw - mu
    var = jnp.mean(hc * hc, axis=0, keepdims=True)
    return hc * lax.rsqrt(var + 1e-5) * g_ref[0, :] + be_ref[0, :]


def _layer_next_body(y_ref, parts_ref, eps_ref, b1_ref, w2_ref, b2_ref, g_ref,
                     be_ref, wn_ref, h_ref, yn_ref):
    hn = _layer_math(y_ref, parts_ref, eps_ref, b1_ref, w2_ref, b2_ref, g_ref,
                     be_ref)
    h_ref[...] = hn
    yn_ref[...] = jnp.dot(hn, wn_ref[...],
                          preferred_element_type=jnp.float32)


def _layer_next(y, parts, eps, b1, w2, b2, g, be, wn):
    return pl.pallas_call(
        _layer_next_body,
        out_shape=[jax.ShapeDtypeStruct((N, H), jnp.float32),
                   jax.ShapeDtypeStruct((N, H), jnp.float32)],
    )(y, parts, eps, b1, w2, b2, g, be, wn)


def _head_body(h1_ref, h2_ref, y3_ref, parts_ref, eps_ref, b1_ref, w2_ref,
               b2_ref, g_ref, be_ref, batch_ref, f1w_ref, f1b_ref, f2w_ref,
               f2b_ref, f3w_ref, f3b_ref, f4w_ref, f4b_ref, out_ref):
    h3 = _layer_math(y3_ref, parts_ref, eps_ref, b1_ref, w2_ref, b2_ref,
                     g_ref, be_ref)
    bt = batch_ref[0, :]
    oh = (lax.broadcasted_iota(jnp.int32, (G, N), 0)
          == bt[None, :]).astype(jnp.float32)
    s1 = jnp.dot(oh, h1_ref[...], preferred_element_type=jnp.float32)
    s2 = jnp.dot(oh, h2_ref[...], preferred_element_type=jnp.float32)
    s3 = jnp.dot(oh, h3, preferred_element_type=jnp.float32)
    cnt = jnp.sum(oh, axis=1, keepdims=True)
    inv = 1.0 / jnp.maximum(cnt, 1.0)
    h = (jnp.dot(s1 * inv, f1w_ref[0:H, :], preferred_element_type=jnp.float32)
         + jnp.dot(s2 * inv, f1w_ref[H:2 * H, :],
                   preferred_element_type=jnp.float32)
         + jnp.dot(s3 * inv, f1w_ref[2 * H:3 * H, :],
                   preferred_element_type=jnp.float32)
         + f1b_ref[0, :])
    h = jnp.maximum(h, 0.0)
    h = jnp.maximum(
        jnp.dot(h, f2w_ref[...], preferred_element_type=jnp.float32)
        + f2b_ref[0, :], 0.0)
    h = jnp.maximum(
        jnp.dot(h, f3w_ref[...], preferred_element_type=jnp.float32)
        + f3b_ref[0, :], 0.0)
    logits = (jnp.dot(h, f4w_ref[...], preferred_element_type=jnp.float32)
              + f4b_ref[0, :])
    shifted = logits - jnp.max(logits, axis=1, keepdims=True)
    lse = jnp.log(jnp.sum(jnp.exp(shifted), axis=1, keepdims=True))
    out_ref[...] = shifted - lse


def _head(h1, h2, y3, parts, eps, b1, w2, b2, g, be, batch2, f1_w, f1_b,
          f2_w, f2_b, f3_w, f3_b, f4_w, f4_b):
    return pl.pallas_call(
        _head_body,
        out_shape=jax.ShapeDtypeStruct((G, C), jnp.float32),
    )(h1, h2, y3, parts, eps, b1, w2, b2, g, be, batch2, f1_w, f1_b, f2_w,
      f2_b, f3_w, f3_b, f4_w, f4_b)


def kernel(x, edge_index, batch, c1_w1, c1_b1, c1_w2, c1_b2, c1_g, c1_be,
           c1_eps, c2_w1, c2_b1, c2_w2, c2_b2, c2_g, c2_be, c2_eps, c3_w1,
           c3_b1, c3_w2, c3_b2, c3_g, c3_be, c3_eps, f1_w, f1_b, f2_w, f2_b,
           f3_w, f3_b, f4_w, f4_b):
    src2 = jnp.concatenate([edge_index[0], jnp.asarray(_PAD_S)]).reshape(
        NCH, CHUNK)
    dst2 = jnp.concatenate([edge_index[1], jnp.asarray(_PAD_D)]).reshape(
        NCH, CHUNK)
    zblk = jnp.zeros((ZR, H), jnp.float32)
    batch2 = batch.reshape(1, N)

    r1 = lambda v: v.reshape(1, -1)
    e1 = lambda v: v.reshape(1, 1)

    y1 = _proj(x, c1_w1)
    p1 = _agg_partials(y1, src2, dst2, zblk)
    h1, y2 = _layer_next(y1, p1, e1(c1_eps), r1(c1_b1), c1_w2, r1(c1_b2),
                         r1(c1_g), r1(c1_be), c2_w1)
    p2 = _agg_partials(y2, src2, dst2, zblk)
    h2, y3 = _layer_next(y2, p2, e1(c2_eps), r1(c2_b1), c2_w2, r1(c2_b2),
                         r1(c2_g), r1(c2_be), c3_w1)
    p3 = _agg_partials(y3, src2, dst2, zblk)
    return _head(h1, h2, y3, p3, e1(c3_eps), r1(c3_b1), c3_w2, r1(c3_b2),
                 r1(c3_g), r1(c3_be), batch2, f1_w, r1(f1_b), f2_w, r1(f2_b),
                 f3_w, r1(f3_b), f4_w, r1(f4_b))


# speedup vs baseline: 1.0041x; 1.0041x over previous
import functools

import numpy as np
import jax
import jax.numpy as jnp
from jax import lax
from jax.experimental import pallas as pl
from jax.experimental.pallas import tpu as pltpu
from jax.experimental.pallas import tpu_sc as plsc

N = 10000
E = 320000
D = 128
H = 64
G = 64
C = 16

NC = 2
NS = 16
NW = NC * NS
CHUNK = 128
CPW = 80
NCH = NW * CPW
EP = NCH * CHUNK
NP = 10240
ZR = NP // NS

_IT = np.arange(EP - E, dtype=np.int32)
_PAD_S = _IT % N
_PAD_D = N + _IT % (NP - N)


NB = 8
NOUT = CPW // NB


def _sc_agg_body(y_hbm, src_hbm, dst_hbm, z_hbm, out_hbm, sidx, didx, rows,
                 gsem, ssem, agg):
    cid = lax.axis_index("c")
    sid = lax.axis_index("s")
    pltpu.sync_copy(z_hbm, agg.at[pl.ds(sid * ZR, ZR)])
    base = (cid * NS + sid) * CPW
    pltpu.sync_copy(src_hbm.at[pl.ds(base, CPW)], sidx)
    pltpu.sync_copy(dst_hbm.at[pl.ds(base, CPW)], didx)
    plsc.subcore_barrier()

    @pl.loop(0, NOUT)
    def _outer(t):
        descs = []
        for k in range(NB):
            @pl.when(t > 0)
            def _drain():
                pltpu.make_async_copy(z_hbm.at[pl.ds(0, CHUNK)],
                                      rows.at[k], ssem.at[k]).wait()
            descs.append(pltpu.async_copy(y_hbm.at[sidx.at[t * NB + k]],
                                          rows.at[k], gsem.at[k]))
        for k in range(NB):
            descs[k].wait()
            pltpu.async_copy(rows.at[k], agg.at[didx.at[t * NB + k]],
                             ssem.at[k], add=True)

    for k in range(NB):
        pltpu.make_async_copy(z_hbm.at[pl.ds(0, CHUNK)],
                              rows.at[k], ssem.at[k]).wait()
    plsc.subcore_barrier()
    pltpu.sync_copy(agg.at[pl.ds(sid * ZR, ZR)],
                    out_hbm.at[cid, pl.ds(sid * ZR, ZR), pl.ds(0, H)])


@functools.cache
def _sc_agg():
    mesh = plsc.VectorSubcoreMesh(core_axis_name="c", subcore_axis_name="s",
                                  num_cores=NC, num_subcores=NS)
    return pl.kernel(
        _sc_agg_body,
        out_type=jax.ShapeDtypeStruct((NC, NP, 2 * H), jnp.float32),
        mesh=mesh,
        compiler_params=pltpu.CompilerParams(use_tc_tiling_on_sc=False),
        scratch_types=[
            pltpu.VMEM((CPW, CHUNK), jnp.int32),
            pltpu.VMEM((CPW, CHUNK), jnp.int32),
            pltpu.VMEM((NB, CHUNK, H), jnp.float32),
            pltpu.SemaphoreType.DMA((NB,)),
            pltpu.SemaphoreType.DMA((NB,)),
            pltpu.VMEM_SHARED((NP, H), jnp.float32),
        ],
    )


def _agg_partials(y, src2, dst2, zblk):
    return _sc_agg()(y, src2, dst2, zblk)


def _proj_body(x_ref, w_ref, o_ref):
    o_ref[...] = jnp.dot(x_ref[...], w_ref[...],
                         preferred_element_type=jnp.float32)


def _proj(x, w):
    return pl.pallas_call(
        _proj_body,
        out_shape=jax.ShapeDtypeStruct((N, H), jnp.float32),
    )(x, w)


def _layer_math(y_ref, parts_ref, eps_ref, b1_ref, w2_ref, b2_ref, g_ref,
                be_ref):
    agg = parts_ref[0, :N, 0:H] + parts_ref[1, :N, 0:H]
    t = jnp.maximum(
        (1.0 + eps_ref[0, 0]) * y_ref[...] + agg + b1_ref[0, :], 0.0)
    hraw = jnp.maximum(
        jnp.dot(t, w2_ref[...], preferred_element_type=jnp.float32)
        + b2_ref[0, :], 0.0)
    mu = jnp.mean(hraw, axis=0, keepdims=True)
    hc = hraw - mu
    var = jnp.mean(hc * hc, axis=0, keepdims=True)
    return hc * lax.rsqrt(var + 1e-5) * g_ref[0, :] + be_ref[0, :]


def _layer_next_body(y_ref, parts_ref, eps_ref, b1_ref, w2_ref, b2_ref, g_ref,
                     be_ref, wn_ref, h_ref, yn_ref):
    hn = _layer_math(y_ref, parts_ref, eps_ref, b1_ref, w2_ref, b2_ref, g_ref,
                     be_ref)
    h_ref[...] = hn
    yn_ref[...] = jnp.dot(hn, wn_ref[...],
                          preferred_element_type=jnp.float32)


def _layer_next(y, parts, eps, b1, w2, b2, g, be, wn):
    return pl.pallas_call(
        _layer_next_body,
        out_shape=[jax.ShapeDtypeStruct((N, H), jnp.float32),
                   jax.ShapeDtypeStruct((N, H), jnp.float32)],
    )(y, parts, eps, b1, w2, b2, g, be, wn)


def _pool_body(h_ref, batch_ref, s_ref, cnt_ref):
    bt = batch_ref[0, :]
    oh = (lax.broadcasted_iota(jnp.int32, (G, N), 0)
          == bt[None, :]).astype(jnp.float32)
    s_ref[...] = jnp.dot(oh, h_ref[...], preferred_element_type=jnp.float32)
    cnt_ref[...] = jnp.sum(oh, axis=1, keepdims=True)


def _pool(h, batch2):
    return pl.pallas_call(
        _pool_body,
        out_shape=[jax.ShapeDtypeStruct((G, H), jnp.float32),
                   jax.ShapeDtypeStruct((G, 1), jnp.float32)],
    )(h, batch2)


def _head_body(s1_ref, s2_ref, cnt_ref, y3_ref, parts_ref, eps_ref, b1_ref,
               w2_ref, b2_ref, g_ref, be_ref, batch_ref, f1w_ref, f1b_ref,
               f2w_ref, f2b_ref, f3w_ref, f3b_ref, f4w_ref, f4b_ref, out_ref):
    h3 = _layer_math(y3_ref, parts_ref, eps_ref, b1_ref, w2_ref, b2_ref,
                     g_ref, be_ref)
    bt = batch_ref[0, :]
    oh = (lax.broadcasted_iota(jnp.int32, (G, N), 0)
          == bt[None, :]).astype(jnp.float32)
    s1 = s1_ref[...]
    s2 = s2_ref[...]
    s3 = jnp.dot(oh, h3, preferred_element_type=jnp.float32)
    inv = 1.0 / jnp.maximum(cnt_ref[...], 1.0)
    h = (jnp.dot(s1 * inv, f1w_ref[0:H, :], preferred_element_type=jnp.float32)
         + jnp.dot(s2 * inv, f1w_ref[H:2 * H, :],
                   preferred_element_type=jnp.float32)
         + jnp.dot(s3 * inv, f1w_ref[2 * H:3 * H, :],
                   preferred_element_type=jnp.float32)
         + f1b_ref[0, :])
    h = jnp.maximum(h, 0.0)
    h = jnp.maximum(
        jnp.dot(h, f2w_ref[...], preferred_element_type=jnp.float32)
        + f2b_ref[0, :], 0.0)
    h = jnp.maximum(
        jnp.dot(h, f3w_ref[...], preferred_element_type=jnp.float32)
        + f3b_ref[0, :], 0.0)
    logits = (jnp.dot(h, f4w_ref[...], preferred_element_type=jnp.float32)
              + f4b_ref[0, :])
    shifted = logits - jnp.max(logits, axis=1, keepdims=True)
    lse = jnp.log(jnp.sum(jnp.exp(shifted), axis=1, keepdims=True))
    out_ref[...] = shifted - lse


def _head(s1, s2, cnt, y3, parts, eps, b1, w2, b2, g, be, batch2, f1_w,
          f1_b, f2_w, f2_b, f3_w, f3_b, f4_w, f4_b):
    return pl.pallas_call(
        _head_body,
        out_shape=jax.ShapeDtypeStruct((G, C), jnp.float32),
    )(s1, s2, cnt, y3, parts, eps, b1, w2, b2, g, be, batch2, f1_w, f1_b,
      f2_w, f2_b, f3_w, f3_b, f4_w, f4_b)


def kernel(x, edge_index, batch, c1_w1, c1_b1, c1_w2, c1_b2, c1_g, c1_be,
           c1_eps, c2_w1, c2_b1, c2_w2, c2_b2, c2_g, c2_be, c2_eps, c3_w1,
           c3_b1, c3_w2, c3_b2, c3_g, c3_be, c3_eps, f1_w, f1_b, f2_w, f2_b,
           f3_w, f3_b, f4_w, f4_b):
    src2 = jnp.concatenate([edge_index[0], jnp.asarray(_PAD_S)]).reshape(
        NCH, CHUNK)
    dst2 = jnp.concatenate([edge_index[1], jnp.asarray(_PAD_D)]).reshape(
        NCH, CHUNK)
    zblk = jnp.zeros((ZR, H), jnp.float32)
    batch2 = batch.reshape(1, N)

    r1 = lambda v: v.reshape(1, -1)
    e1 = lambda v: v.reshape(1, 1)

    y1 = _proj(x, c1_w1)
    p1 = _agg_partials(y1, src2, dst2, zblk)
    h1, y2 = _layer_next(y1, p1, e1(c1_eps), r1(c1_b1), c1_w2, r1(c1_b2),
                         r1(c1_g), r1(c1_be), c2_w1)
    p2 = _agg_partials(y2, src2, dst2, zblk)
    s1, cnt = _pool(h1, batch2)
    h2, y3 = _layer_next(y2, p2, e1(c2_eps), r1(c2_b1), c2_w2, r1(c2_b2),
                         r1(c2_g), r1(c2_be), c3_w1)
    p3 = _agg_partials(y3, src2, dst2, zblk)
    s2, _ = _pool(h2, batch2)
    return _head(s1, s2, cnt, y3, p3, e1(c3_eps), r1(c3_b1), c3_w2, r1(c3_b2),
                 r1(c3_g), r1(c3_be), batch2, f1_w, r1(f1_b), f2_w, r1(f2_b),
                 f3_w, r1(f3_b), f4_w, r1(f4_b))
